# Initial kernel scaffold; baseline (speedup 1.0000x reference)
import jax
import jax.numpy as jnp
from jax.experimental import pallas as pl


def kernel(embed_user, embed_item, user_item_matrix, item_user_matrix, d_i_train, d_j_train, user, item_i, item_j):
    raise NotImplementedError("write your pallas kernel here")


# speedup vs baseline: 1.2954x; 1.2954x over previous
import jax
import jax.numpy as jnp
from jax.experimental import pallas as pl
from jax.experimental.pallas import tpu as pltpu


def _gcn_kernel(a_ref, b_ref, eu_ref, ei_ref, di_ref, dj_ref,
                tu_ref, ti_ref):
    a = a_ref[...].astype(jnp.bfloat16)
    b = b_ref[...].astype(jnp.bfloat16)
    eu = eu_ref[...]
    ei = ei_ref[...]
    di = di_ref[...]
    dj = dj_ref[...]

    def prop(adj, other, self_, d_col):
        return jnp.dot(adj, other.astype(jnp.bfloat16),
                       preferred_element_type=jnp.float32) + self_ * d_col

    g1u = prop(a, ei, eu, di)
    g1i = prop(b, eu, ei, dj)
    g2u = prop(a, g1i, g1u, di)
    g2i = prop(b, g1u, g1i, dj)
    g3u = prop(a, g2i, g2u, di)
    g3i = prop(b, g2u, g2i, dj)

    f = eu.shape[1]
    tu_ref[:, 0 * f:1 * f] = eu
    tu_ref[:, 1 * f:2 * f] = g1u
    tu_ref[:, 2 * f:3 * f] = g2u
    tu_ref[:, 3 * f:4 * f] = g3u
    ti_ref[:, 0 * f:1 * f] = ei
    ti_ref[:, 1 * f:2 * f] = g1i
    ti_ref[:, 2 * f:3 * f] = g2i
    ti_ref[:, 3 * f:4 * f] = g3i


def _bpr_kernel(tu_ref, ti_ref, u_ref, i_ref, j_ref,
                pi_ref, pj_ref, logp_ref, l2p_ref):
    tu = tu_ref[...]
    ti = ti_ref[...]
    num_users = tu.shape[0]
    num_items = ti.shape[0]
    tb = u_ref.shape[0]

    def take(table, idx_col, n):
        onehot = (jax.lax.broadcasted_iota(jnp.int32, (tb, n), 1)
                  == idx_col).astype(jnp.float32)
        return jnp.dot(onehot, table, preferred_element_type=jnp.float32)

    u = take(tu, u_ref[...], num_users)
    vi = take(ti, i_ref[...], num_items)
    vj = take(ti, j_ref[...], num_items)

    pi = jnp.sum(u * vi, axis=1, keepdims=True)
    pj = jnp.sum(u * vj, axis=1, keepdims=True)
    l2 = 0.01 * jnp.sum(u * u + vi * vi + vj * vj, axis=1, keepdims=True)

    diff = pi - pj
    log_sig = jnp.minimum(diff, 0.0) - jnp.log(1.0 + jnp.exp(-jnp.abs(diff)))

    pi_ref[...] = pi
    pj_ref[...] = pj
    logp_ref[...] = jnp.sum(log_sig).reshape(1, 1, 1)
    l2p_ref[...] = jnp.sum(l2).reshape(1, 1, 1)


def kernel(embed_user, embed_item, user_item_matrix, item_user_matrix,
           d_i_train, d_j_train, user, item_i, item_j):
    num_users, factor_num = embed_user.shape
    num_items = embed_item.shape[0]
    d4 = 4 * factor_num
    batch = user.shape[0]

    tu, ti = pl.pallas_call(
        _gcn_kernel,
        out_shape=(
            jax.ShapeDtypeStruct((num_users, d4), jnp.float32),
            jax.ShapeDtypeStruct((num_items, d4), jnp.float32),
        ),
        compiler_params=pltpu.CompilerParams(
            vmem_limit_bytes=56 * 1024 * 1024),
    )(user_item_matrix, item_user_matrix, embed_user, embed_item,
      d_i_train, d_j_train)

    tb = 512
    while batch % tb:
        tb //= 2
    n_tiles = batch // tb

    u_col = user.astype(jnp.int32).reshape(batch, 1)
    i_col = item_i.astype(jnp.int32).reshape(batch, 1)
    j_col = item_j.astype(jnp.int32).reshape(batch, 1)

    pi, pj, logp, l2p = pl.pallas_call(
        _bpr_kernel,
        out_shape=(
            jax.ShapeDtypeStruct((batch, 1), jnp.float32),
            jax.ShapeDtypeStruct((batch, 1), jnp.float32),
            jax.ShapeDtypeStruct((n_tiles, 1, 1), jnp.float32),
            jax.ShapeDtypeStruct((n_tiles, 1, 1), jnp.float32),
        ),
        grid=(n_tiles,),
        in_specs=[
            pl.BlockSpec((num_users, d4), lambda t: (0, 0)),
            pl.BlockSpec((num_items, d4), lambda t: (0, 0)),
            pl.BlockSpec((tb, 1), lambda t: (t, 0)),
            pl.BlockSpec((tb, 1), lambda t: (t, 0)),
            pl.BlockSpec((tb, 1), lambda t: (t, 0)),
        ],
        out_specs=(
            pl.BlockSpec((tb, 1), lambda t: (t, 0)),
            pl.BlockSpec((tb, 1), lambda t: (t, 0)),
            pl.BlockSpec((1, 1, 1), lambda t: (t, 0, 0)),
            pl.BlockSpec((1, 1, 1), lambda t: (t, 0, 0)),
        ),
        compiler_params=pltpu.CompilerParams(
            dimension_semantics=("arbitrary",),
            vmem_limit_bytes=56 * 1024 * 1024),
    )(tu, ti, u_col, i_col, j_col)

    loss2 = -jnp.sum(logp) / batch
    loss = loss2 + jnp.sum(l2p) / batch
    return pi.reshape(batch), pj.reshape(batch), loss, loss2


# speedup vs baseline: 1.5540x; 1.1996x over previous
import jax
import jax.numpy as jnp
from jax.experimental import pallas as pl
from jax.experimental.pallas import tpu as pltpu


def _hi_lo(x):
    hi = x.astype(jnp.bfloat16)
    lo = (x - hi.astype(jnp.float32)).astype(jnp.bfloat16)
    return hi, lo


def _gcn_kernel(a_ref, b_ref, eu_ref, ei_ref, di_ref, dj_ref,
                tuh_ref, tul_ref, tih_ref, til_ref):
    a = a_ref[...].astype(jnp.bfloat16)
    b = b_ref[...].astype(jnp.bfloat16)
    eu_t = eu_ref[...].T
    ei_t = ei_ref[...].T
    di_t = di_ref[...].T
    dj_t = dj_ref[...].T

    def prop(other_t, self_t, adj, d_row):
        acc = jax.lax.dot_general(
            other_t.astype(jnp.bfloat16), adj,
            (((1,), (1,)), ((), ())),
            preferred_element_type=jnp.float32)
        return acc + self_t * d_row

    g1u = prop(ei_t, eu_t, a, di_t)
    g1i = prop(eu_t, ei_t, b, dj_t)
    g2u = prop(g1i, g1u, a, di_t)
    g2i = prop(g1u, g1i, b, dj_t)
    g3u = prop(g2i, g2u, a, di_t)
    g3i = prop(g2u, g2i, b, dj_t)

    f = eu_t.shape[0]
    for k, (gu, gi) in enumerate(((eu_t, ei_t), (g1u, g1i),
                                  (g2u, g2i), (g3u, g3i))):
        hu, lu = _hi_lo(gu)
        hi_, li_ = _hi_lo(gi)
        tuh_ref[k * f:(k + 1) * f, :] = hu
        tul_ref[k * f:(k + 1) * f, :] = lu
        tih_ref[k * f:(k + 1) * f, :] = hi_
        til_ref[k * f:(k + 1) * f, :] = li_


def _bpr_kernel(tuh_ref, tul_ref, tih_ref, til_ref, u_ref, i_ref, j_ref,
                pi_ref, pj_ref, logp_ref, l2p_ref):
    num_users = tuh_ref.shape[1]
    num_items = tih_ref.shape[1]
    tb = u_ref.shape[1]

    def take(hi_t, lo_t, idx_row, n):
        onehot = (jax.lax.broadcasted_iota(jnp.int32, (n, tb), 0)
                  == idx_row).astype(jnp.bfloat16)
        return (jnp.dot(hi_t, onehot, preferred_element_type=jnp.float32)
                + jnp.dot(lo_t, onehot, preferred_element_type=jnp.float32))

    u = take(tuh_ref[...], tul_ref[...], u_ref[...], num_users)
    vi = take(tih_ref[...], til_ref[...], i_ref[...], num_items)
    vj = take(tih_ref[...], til_ref[...], j_ref[...], num_items)

    pi = jnp.sum(u * vi, axis=0, keepdims=True)
    pj = jnp.sum(u * vj, axis=0, keepdims=True)
    l2 = 0.01 * jnp.sum(u * u + vi * vi + vj * vj, axis=0, keepdims=True)

    diff = pi - pj
    log_sig = jnp.minimum(diff, 0.0) - jnp.log(1.0 + jnp.exp(-jnp.abs(diff)))

    pi_ref[...] = pi
    pj_ref[...] = pj
    logp_ref[...] = jnp.sum(log_sig).reshape(1, 1, 1)
    l2p_ref[...] = jnp.sum(l2).reshape(1, 1, 1)


def kernel(embed_user, embed_item, user_item_matrix, item_user_matrix,
           d_i_train, d_j_train, user, item_i, item_j):
    num_users, factor_num = embed_user.shape
    num_items = embed_item.shape[0]
    d4 = 4 * factor_num
    batch = user.shape[0]

    tuh, tul, tih, til = pl.pallas_call(
        _gcn_kernel,
        out_shape=(
            jax.ShapeDtypeStruct((d4, num_users), jnp.bfloat16),
            jax.ShapeDtypeStruct((d4, num_users), jnp.bfloat16),
            jax.ShapeDtypeStruct((d4, num_items), jnp.bfloat16),
            jax.ShapeDtypeStruct((d4, num_items), jnp.bfloat16),
        ),
        compiler_params=pltpu.CompilerParams(
            vmem_limit_bytes=56 * 1024 * 1024),
    )(user_item_matrix, item_user_matrix, embed_user, embed_item,
      d_i_train, d_j_train)

    tb = 512
    while batch % tb:
        tb //= 2
    n_tiles = batch // tb

    u_row = user.astype(jnp.int32).reshape(1, batch)
    i_row = item_i.astype(jnp.int32).reshape(1, batch)
    j_row = item_j.astype(jnp.int32).reshape(1, batch)

    pi, pj, logp, l2p = pl.pallas_call(
        _bpr_kernel,
        out_shape=(
            jax.ShapeDtypeStruct((1, batch), jnp.float32),
            jax.ShapeDtypeStruct((1, batch), jnp.float32),
            jax.ShapeDtypeStruct((n_tiles, 1, 1), jnp.float32),
            jax.ShapeDtypeStruct((n_tiles, 1, 1), jnp.float32),
        ),
        grid=(n_tiles,),
        in_specs=[
            pl.BlockSpec((d4, num_users), lambda t: (0, 0)),
            pl.BlockSpec((d4, num_users), lambda t: (0, 0)),
            pl.BlockSpec((d4, num_items), lambda t: (0, 0)),
            pl.BlockSpec((d4, num_items), lambda t: (0, 0)),
            pl.BlockSpec((1, tb), lambda t: (0, t)),
            pl.BlockSpec((1, tb), lambda t: (0, t)),
            pl.BlockSpec((1, tb), lambda t: (0, t)),
        ],
        out_specs=(
            pl.BlockSpec((1, tb), lambda t: (0, t)),
            pl.BlockSpec((1, tb), lambda t: (0, t)),
            pl.BlockSpec((1, 1, 1), lambda t: (t, 0, 0)),
            pl.BlockSpec((1, 1, 1), lambda t: (t, 0, 0)),
        ),
        compiler_params=pltpu.CompilerParams(
            dimension_semantics=("arbitrary",),
            vmem_limit_bytes=56 * 1024 * 1024),
    )(tuh, tul, tih, til, u_row, i_row, j_row)

    loss2 = -jnp.sum(logp) / batch
    loss = loss2 + jnp.sum(l2p) / batch
    return pi.reshape(batch), pj.reshape(batch), loss, loss2


# speedup vs baseline: 1.6286x; 1.0481x over previous
import jax
import jax.numpy as jnp
from jax.experimental import pallas as pl
from jax.experimental.pallas import tpu as pltpu


def _hi_lo(x):
    hi = x.astype(jnp.bfloat16)
    lo = (x - hi.astype(jnp.float32)).astype(jnp.bfloat16)
    return hi, lo


def _fused_kernel(a_ref, b_ref, eu_ref, ei_ref, di_ref, dj_ref,
                  u_ref, i_ref, j_ref,
                  pi_ref, pj_ref, logp_ref, l2p_ref,
                  tuh_s, tul_s, tih_s, til_s):
    t = pl.program_id(0)

    @pl.when(t == 0)
    def _gcn():
        a = a_ref[...].astype(jnp.bfloat16)
        b = b_ref[...].astype(jnp.bfloat16)
        eu_t = eu_ref[...].T
        ei_t = ei_ref[...].T
        di_t = di_ref[...].T
        dj_t = dj_ref[...].T

        def prop(other_t, self_t, adj, d_row):
            acc = jax.lax.dot_general(
                other_t.astype(jnp.bfloat16), adj,
                (((1,), (1,)), ((), ())),
                preferred_element_type=jnp.float32)
            return acc + self_t * d_row

        g1u = prop(ei_t, eu_t, a, di_t)
        g1i = prop(eu_t, ei_t, b, dj_t)
        g2u = prop(g1i, g1u, a, di_t)
        g2i = prop(g1u, g1i, b, dj_t)
        g3u = prop(g2i, g2u, a, di_t)
        g3i = prop(g2u, g2i, b, dj_t)

        f = eu_t.shape[0]
        for k, (gu, gi) in enumerate(((eu_t, ei_t), (g1u, g1i),
                                      (g2u, g2i), (g3u, g3i))):
            hu, lu = _hi_lo(gu)
            hi_, li_ = _hi_lo(gi)
            tuh_s[k * f:(k + 1) * f, :] = hu
            tul_s[k * f:(k + 1) * f, :] = lu
            tih_s[k * f:(k + 1) * f, :] = hi_
            til_s[k * f:(k + 1) * f, :] = li_

    @pl.when(t > 0)
    def _bpr():
        num_users = tuh_s.shape[1]
        num_items = tih_s.shape[1]
        tb = u_ref.shape[1]

        def take(hi_t, lo_t, idx_row, n):
            onehot = (jax.lax.broadcasted_iota(jnp.int32, (n, tb), 0)
                      == idx_row).astype(jnp.bfloat16)
            return (jnp.dot(hi_t, onehot, preferred_element_type=jnp.float32)
                    + jnp.dot(lo_t, onehot,
                              preferred_element_type=jnp.float32))

        u = take(tuh_s[...], tul_s[...], u_ref[...], num_users)
        vi = take(tih_s[...], til_s[...], i_ref[...], num_items)
        vj = take(tih_s[...], til_s[...], j_ref[...], num_items)

        pi = jnp.sum(u * vi, axis=0, keepdims=True)
        pj = jnp.sum(u * vj, axis=0, keepdims=True)
        l2 = 0.01 * jnp.sum(u * u + vi * vi + vj * vj,
                            axis=0, keepdims=True)

        diff = pi - pj
        log_sig = (jnp.minimum(diff, 0.0)
                   - jnp.log(1.0 + jnp.exp(-jnp.abs(diff))))

        pi_ref[...] = pi
        pj_ref[...] = pj
        logp_ref[...] = jnp.sum(log_sig).reshape(1, 1, 1)
        l2p_ref[...] = jnp.sum(l2).reshape(1, 1, 1)


def kernel(embed_user, embed_item, user_item_matrix, item_user_matrix,
           d_i_train, d_j_train, user, item_i, item_j):
    num_users, factor_num = embed_user.shape
    num_items = embed_item.shape[0]
    d4 = 4 * factor_num
    batch = user.shape[0]

    tb = 512
    while batch % tb:
        tb //= 2
    n_tiles = batch // tb

    u_row = user.astype(jnp.int32).reshape(1, batch)
    i_row = item_i.astype(jnp.int32).reshape(1, batch)
    j_row = item_j.astype(jnp.int32).reshape(1, batch)

    def tile_idx(t):
        return (0, jnp.maximum(t - 1, 0))

    pi, pj, logp, l2p = pl.pallas_call(
        _fused_kernel,
        out_shape=(
            jax.ShapeDtypeStruct((1, batch), jnp.float32),
            jax.ShapeDtypeStruct((1, batch), jnp.float32),
            jax.ShapeDtypeStruct((n_tiles, 1, 1), jnp.float32),
            jax.ShapeDtypeStruct((n_tiles, 1, 1), jnp.float32),
        ),
        grid=(n_tiles + 1,),
        in_specs=[
            pl.BlockSpec((num_users, num_items), lambda t: (0, 0)),
            pl.BlockSpec((num_items, num_users), lambda t: (0, 0)),
            pl.BlockSpec((num_users, factor_num), lambda t: (0, 0)),
            pl.BlockSpec((num_items, factor_num), lambda t: (0, 0)),
            pl.BlockSpec((num_users, 1), lambda t: (0, 0)),
            pl.BlockSpec((num_items, 1), lambda t: (0, 0)),
            pl.BlockSpec((1, tb), tile_idx),
            pl.BlockSpec((1, tb), tile_idx),
            pl.BlockSpec((1, tb), tile_idx),
        ],
        out_specs=(
            pl.BlockSpec((1, tb), tile_idx),
            pl.BlockSpec((1, tb), tile_idx),
            pl.BlockSpec((1, 1, 1), lambda t: (jnp.maximum(t - 1, 0), 0, 0)),
            pl.BlockSpec((1, 1, 1), lambda t: (jnp.maximum(t - 1, 0), 0, 0)),
        ),
        scratch_shapes=[
            pltpu.VMEM((d4, num_users), jnp.bfloat16),
            pltpu.VMEM((d4, num_users), jnp.bfloat16),
            pltpu.VMEM((d4, num_items), jnp.bfloat16),
            pltpu.VMEM((d4, num_items), jnp.bfloat16),
        ],
        compiler_params=pltpu.CompilerParams(
            dimension_semantics=("arbitrary",),
            vmem_limit_bytes=56 * 1024 * 1024),
    )(user_item_matrix, item_user_matrix, embed_user, embed_item,
      d_i_train, d_j_train, u_row, i_row, j_row)

    loss2 = -jnp.sum(logp) / batch
    loss = loss2 + jnp.sum(l2p) / batch
    return pi.reshape(batch), pj.reshape(batch), loss, loss2
